# baseline (speedup 1.0000x reference)
import functools

import jax
import jax.numpy as jnp
from jax import lax
from jax.experimental import pallas as pl
from jax.experimental.pallas import tpu as pltpu
from jax.experimental.pallas import tpu_sc as plsc

_LANES = 16
_NUM_WORKERS = 32
_CHUNK_ROWS = 8
_UNROLL = 8


def _lookup_sc(x, table_len):
    rows, cols = x.shape
    per_worker_rows = rows // _NUM_WORKERS
    n_chunks = per_worker_rows // _CHUNK_ROWS
    assert rows % _NUM_WORKERS == 0 and per_worker_rows % _CHUNK_ROWS == 0
    assert n_chunks % 2 == 0 and cols % _LANES == 0
    vecs_per_row = cols // _LANES
    steps = _CHUNK_ROWS * vecs_per_row
    fmax = float(table_len - 1)

    mesh = plsc.VectorSubcoreMesh(core_axis_name="c", subcore_axis_name="s")

    @functools.partial(
        pl.kernel,
        out_type=jax.ShapeDtypeStruct((rows, cols), jnp.float32),
        mesh=mesh,
        compiler_params=pltpu.CompilerParams(needs_layout_passes=False),
        scratch_types=[
            pltpu.VMEM((table_len,), jnp.float32),
            pltpu.VMEM((_LANES,), jnp.float32),
            pltpu.VMEM((_LANES,), jnp.float32),
            pltpu.VMEM((_CHUNK_ROWS, cols), jnp.float32),
            pltpu.VMEM((_CHUNK_ROWS, cols), jnp.float32),
            pltpu.VMEM((_CHUNK_ROWS, cols), jnp.float32),
            pltpu.VMEM((_CHUNK_ROWS, cols), jnp.float32),
            pltpu.VMEM((_CHUNK_ROWS, cols), jnp.float32),
            pltpu.VMEM((_CHUNK_ROWS, cols), jnp.float32),
            pltpu.SemaphoreType.DMA,
            pltpu.SemaphoreType.DMA,
            pltpu.SemaphoreType.DMA,
            pltpu.SemaphoreType.DMA,
            pltpu.SemaphoreType.DMA,
            pltpu.SemaphoreType.DMA,
        ],
    )
    def body(x_hbm, tab_hbm, scale_hbm, min_hbm, out_hbm,
             tab_v, scale_v, min_v, in0, in1, in2, ob0, ob1, ob2,
             si0, si1, si2, so0, so1, so2):
        ins = (in0, in1, in2)
        obs = (ob0, ob1, ob2)
        sis = (si0, si1, si2)
        sos = (so0, so1, so2)

        wid = lax.axis_index("s") * 2 + lax.axis_index("c")
        base_row = wid * per_worker_rows

        pltpu.sync_copy(tab_hbm, tab_v)
        pltpu.sync_copy(scale_hbm, scale_v)
        pltpu.sync_copy(min_hbm, min_v)
        scale = scale_v[...]
        minv = min_v[...]

        def start_in(c, b):
            pltpu.async_copy(
                x_hbm.at[pl.ds(base_row + c * _CHUNK_ROWS, _CHUNK_ROWS), :],
                ins[b], sis[b])

        def wait_in(b):
            pltpu.make_async_copy(
                x_hbm.at[pl.ds(0, _CHUNK_ROWS), :], ins[b], sis[b]).wait()

        def start_out(c, b):
            pltpu.async_copy(
                obs[b],
                out_hbm.at[pl.ds(base_row + c * _CHUNK_ROWS, _CHUNK_ROWS), :],
                sos[b])

        def wait_out(b):
            pltpu.make_async_copy(
                obs[b], out_hbm.at[pl.ds(0, _CHUNK_ROWS), :], sos[b]).wait()

        def compute(b):
            src = ins[b]
            dst = obs[b]

            for r in range(_CHUNK_ROWS):
                @plsc.parallel_loop(0, vecs_per_row, unroll=_UNROLL)
                def _steps(i, r=r):
                    off = i * _LANES
                    v = src[r, pl.ds(off, _LANES)]
                    q = jnp.clip((v - minv) * scale, 0.0, fmax)
                    idx = q.astype(jnp.int32)
                    dst[r, pl.ds(off, _LANES)] = plsc.load_gather(tab_v, [idx])

        nbuf = len(ins)
        rounds = n_chunks // nbuf
        leftover = n_chunks - rounds * nbuf

        for b in range(nbuf):
            start_in(b, b)

        @pl.loop(0, rounds)
        def _rounds(r2):
            for b in range(nbuf):
                c = r2 * nbuf + b
                wait_in(b)

                @pl.when(r2 > 0)
                def _():
                    wait_out(b)

                compute(b)
                start_out(c, b)

                @pl.when(c + nbuf < n_chunks)
                def _():
                    start_in(c + nbuf, b)

        for i in range(leftover):
            c = rounds * nbuf + i
            wait_in(i)
            wait_out(i)
            compute(i)
            start_out(c, i)

        for b in range(nbuf):
            wait_out(b)

    return body


def kernel(x, forward_values, backward_values, input_min, input_max):
    del backward_values
    table_len = forward_values.shape[0]
    idx_max = table_len - 1
    scale = jnp.float32(idx_max) / (
        jnp.asarray(input_max, jnp.float32) - jnp.asarray(input_min, jnp.float32))
    scale16 = jnp.full((_LANES,), scale, jnp.float32)
    min16 = jnp.full((_LANES,), jnp.asarray(input_min, jnp.float32))

    fn = _lookup_sc(x, table_len)
    return fn(x, forward_values.astype(jnp.float32), scale16, min16)


# speedup vs baseline: 1.0845x; 1.0845x over previous
import functools

import jax
import jax.numpy as jnp
from jax import lax
from jax.experimental import pallas as pl
from jax.experimental.pallas import tpu as pltpu
from jax.experimental.pallas import tpu_sc as plsc

_LANES = 16
_NUM_WORKERS = 32
_CHUNK_ROWS = 8
_UNROLL = 8


def _lookup_sc(x, table_len):
    rows, cols = x.shape
    per_worker_rows = rows // _NUM_WORKERS
    n_chunks = per_worker_rows // _CHUNK_ROWS
    assert rows % _NUM_WORKERS == 0 and per_worker_rows % _CHUNK_ROWS == 0
    assert n_chunks % 2 == 0 and cols % _LANES == 0
    vecs_per_row = cols // _LANES
    steps = _CHUNK_ROWS * vecs_per_row
    fmax = float(table_len - 1)

    mesh = plsc.VectorSubcoreMesh(core_axis_name="c", subcore_axis_name="s")

    @functools.partial(
        pl.kernel,
        out_type=jax.ShapeDtypeStruct((rows, cols), jnp.float32),
        mesh=mesh,
        compiler_params=pltpu.CompilerParams(needs_layout_passes=False),
        scratch_types=[
            pltpu.VMEM((table_len,), jnp.float32),
            pltpu.VMEM((_LANES,), jnp.float32),
            pltpu.VMEM((_LANES,), jnp.float32),
            pltpu.VMEM((_CHUNK_ROWS, cols), jnp.float32),
            pltpu.VMEM((_CHUNK_ROWS, cols), jnp.float32),
            pltpu.VMEM((_CHUNK_ROWS, cols), jnp.float32),
            pltpu.VMEM((_CHUNK_ROWS, cols), jnp.float32),
            pltpu.SemaphoreType.DMA,
            pltpu.SemaphoreType.DMA,
            pltpu.SemaphoreType.DMA,
            pltpu.SemaphoreType.DMA,
        ],
    )
    def body(x_hbm, tab_hbm, scale_hbm, min_hbm, out_hbm,
             tab_v, scale_v, min_v, in0, in1, ob0, ob1,
             si0, si1, so0, so1):
        ins = (in0, in1)
        obs = (ob0, ob1)
        sis = (si0, si1)
        sos = (so0, so1)

        wid = lax.axis_index("s") * 2 + lax.axis_index("c")
        base_row = wid * per_worker_rows

        pltpu.sync_copy(tab_hbm, tab_v)
        pltpu.sync_copy(scale_hbm, scale_v)
        pltpu.sync_copy(min_hbm, min_v)
        scale = scale_v[...]
        minv = min_v[...]

        def start_in(c, b):
            pltpu.async_copy(
                x_hbm.at[pl.ds(base_row + c * _CHUNK_ROWS, _CHUNK_ROWS), :],
                ins[b], sis[b])

        def wait_in(b):
            pltpu.make_async_copy(
                x_hbm.at[pl.ds(0, _CHUNK_ROWS), :], ins[b], sis[b]).wait()

        def start_out(c, b):
            pltpu.async_copy(
                obs[b],
                out_hbm.at[pl.ds(base_row + c * _CHUNK_ROWS, _CHUNK_ROWS), :],
                sos[b])

        def wait_out(b):
            pltpu.make_async_copy(
                obs[b], out_hbm.at[pl.ds(0, _CHUNK_ROWS), :], sos[b]).wait()

        def compute(b):
            src = ins[b]
            dst = obs[b]

            for r in range(_CHUNK_ROWS):
                @plsc.parallel_loop(0, vecs_per_row, unroll=_UNROLL)
                def _steps(i, r=r):
                    off = i * _LANES
                    v = src[r, pl.ds(off, _LANES)]
                    q = jnp.clip((v - minv) * scale, 0.0, fmax)
                    idx = q.astype(jnp.int32)
                    dst[r, pl.ds(off, _LANES)] = plsc.load_gather(tab_v, [idx])

        nbuf = len(ins)
        rounds = n_chunks // nbuf
        leftover = n_chunks - rounds * nbuf

        for b in range(nbuf):
            start_in(b, b)

        @pl.loop(0, rounds)
        def _rounds(r2):
            for b in range(nbuf):
                c = r2 * nbuf + b
                wait_in(b)

                @pl.when(r2 > 0)
                def _():
                    wait_out(b)

                compute(b)
                start_out(c, b)

                @pl.when(c + nbuf < n_chunks)
                def _():
                    start_in(c + nbuf, b)

        for i in range(leftover):
            c = rounds * nbuf + i
            wait_in(i)
            wait_out(i)
            compute(i)
            start_out(c, i)

        for b in range(nbuf):
            wait_out(b)

    return body


def kernel(x, forward_values, backward_values, input_min, input_max):
    del backward_values
    table_len = forward_values.shape[0]
    idx_max = table_len - 1
    scale = jnp.float32(idx_max) / (
        jnp.asarray(input_max, jnp.float32) - jnp.asarray(input_min, jnp.float32))
    scale16 = jnp.full((_LANES,), scale, jnp.float32)
    min16 = jnp.full((_LANES,), jnp.asarray(input_min, jnp.float32))

    fn = _lookup_sc(x, table_len)
    return fn(x, forward_values.astype(jnp.float32), scale16, min16)
